# baseline (device time: 52518 ns/iter reference)
import jax
import jax.numpy as jnp
from jax import lax
from jax.experimental import pallas as pl
from jax.experimental.pallas import tpu as pltpu

N_DEV = 4
NP = 4


def kernel(A, B):
    m, _ = A.shape
    _, n = B.shape
    chunk = m // N_DEV
    half = n // 2
    piece = chunk // NP

    def body(
        a_ref,
        b_ref,
        out_ref,
        b_bf,
        pcL, pcR,
        rs_send_r, rs_recv_r, rs_send_l, rs_recv_l,
        ag_own_r, ag_recv_r, ag_own_l, ag_recv_l,
        rs_ssem_r, rs_rsem_r, rs_ssem_l, rs_rsem_l,
        ag_ssem_r, ag_rsem_r, ag_ssem_l, ag_rsem_l,
    ):
        my = lax.axis_index("i")
        left = (my - 1) % N_DEV
        right = (my + 1) % N_DEV

        barrier_sem = pltpu.get_barrier_semaphore()
        for nbr in (left, right):
            pl.semaphore_signal(
                barrier_sem,
                inc=1,
                device_id=(nbr,),
                device_id_type=pl.DeviceIdType.MESH,
            )
        pl.semaphore_wait(barrier_sem, 2)

        b_bf[...] = b_ref[...].astype(jnp.bfloat16)

        def rdma(src, dst, ssem, rsem, target):
            return pltpu.make_async_remote_copy(
                src_ref=src, dst_ref=dst, send_sem=ssem, recv_sem=rsem,
                device_id=(target,), device_id_type=pl.DeviceIdType.MESH,
            )

        rs_r = [[rdma(rs_send_r.at[s, p], rs_recv_r.at[s, p],
                      rs_ssem_r.at[s, p], rs_rsem_r.at[s, p], right)
                 for p in range(NP)] for s in range(3)]
        rs_l = [[rdma(rs_send_l.at[s, p], rs_recv_l.at[s, p],
                      rs_ssem_l.at[s, p], rs_rsem_l.at[s, p], left)
                 for p in range(NP)] for s in range(3)]
        ag_r = [[rdma(ag_own_r.at[p] if s == 0 else ag_recv_r.at[s - 1, p],
                      ag_recv_r.at[s, p], ag_ssem_r.at[s, p],
                      ag_rsem_r.at[s, p], right)
                 for p in range(NP)] for s in range(3)]
        ag_l = [[rdma(ag_own_l.at[p] if s == 0 else ag_recv_l.at[s - 1, p],
                      ag_recv_l.at[s, p], ag_ssem_l.at[s, p],
                      ag_rsem_l.at[s, p], left)
                 for p in range(NP)] for s in range(3)]

        def a_rows(c, p):
            return a_ref[
                pl.ds(c * chunk + p * piece, piece), :
            ].astype(jnp.bfloat16)

        for p in range(NP):
            ap = a_rows(my, p)
            rs_send_r[0, p, :, :] = jnp.dot(
                ap, b_bf[:, :half], preferred_element_type=jnp.float32
            ).astype(jnp.bfloat16)
            rs_r[0][p].start()
            rs_send_l[0, p, :, :] = jnp.dot(
                ap, b_bf[:, half:], preferred_element_type=jnp.float32
            ).astype(jnp.bfloat16)
            rs_l[0][p].start()

        for c in ((my - 1) % N_DEV, (my + 1) % N_DEV, (my + 2) % N_DEV):
            ac = a_ref[pl.ds(c * chunk, chunk), :].astype(jnp.bfloat16)
            pcL[pl.ds(c * chunk, chunk), :] = jnp.dot(
                ac, b_bf[:, :half], preferred_element_type=jnp.float32
            ).astype(jnp.bfloat16)
            pcR[pl.ds(c * chunk, chunk), :] = jnp.dot(
                ac, b_bf[:, half:], preferred_element_type=jnp.float32
            ).astype(jnp.bfloat16)

        for s in range(2):
            cr = (my - s - 1) % N_DEV
            cl = (my + s + 1) % N_DEV
            for p in range(NP):
                rs_r[s][p].wait_recv()
                rs_send_r[s + 1, p, :, :] = (
                    pcL[pl.ds(cr * chunk + p * piece, piece), :]
                    + rs_recv_r[s, p, :, :]
                )
                rs_r[s + 1][p].start()
                rs_l[s][p].wait_recv()
                rs_send_l[s + 1, p, :, :] = (
                    pcR[pl.ds(cl * chunk + p * piece, piece), :]
                    + rs_recv_l[s, p, :, :]
                )
                rs_l[s + 1][p].start()

        own_r = (my + 1) % N_DEV
        own_l = (my - 1) % N_DEV
        for p in range(NP):
            rs_r[2][p].wait_recv()
            zr = pcL[
                pl.ds(own_r * chunk + p * piece, piece), :
            ].astype(jnp.float32) + rs_recv_r[2, p, :, :].astype(jnp.float32)
            silu_r = zr / (1.0 + jnp.exp(-zr))
            ag_own_r[p, :, :] = silu_r.astype(jnp.bfloat16)
            ag_r[0][p].start()
            out_ref[
                pl.ds(own_r * chunk + p * piece, piece), pl.ds(0, half)
            ] = silu_r
            rs_l[2][p].wait_recv()
            zl = pcR[
                pl.ds(own_l * chunk + p * piece, piece), :
            ].astype(jnp.float32) + rs_recv_l[2, p, :, :].astype(jnp.float32)
            silu_l = zl / (1.0 + jnp.exp(-zl))
            ag_own_l[p, :, :] = silu_l.astype(jnp.bfloat16)
            ag_l[0][p].start()
            out_ref[
                pl.ds(own_l * chunk + p * piece, piece), pl.ds(half, half)
            ] = silu_l

        for s in range(3):
            cr = (my - s) % N_DEV
            cl = (my + s) % N_DEV
            for p in range(NP):
                ag_r[s][p].wait_recv()
                if s < 2:
                    ag_r[s + 1][p].start()
                out_ref[
                    pl.ds(cr * chunk + p * piece, piece), pl.ds(0, half)
                ] = ag_recv_r[s, p, :, :].astype(jnp.float32)
                ag_l[s][p].wait_recv()
                if s < 2:
                    ag_l[s + 1][p].start()
                out_ref[
                    pl.ds(cl * chunk + p * piece, piece), pl.ds(half, half)
                ] = ag_recv_l[s, p, :, :].astype(jnp.float32)

        for grid in (rs_r, rs_l, ag_r, ag_l):
            for ops in grid:
                for op in ops:
                    op.wait_send()

    nhop = N_DEV - 1
    return pl.pallas_call(
        body,
        out_shape=jax.ShapeDtypeStruct((m, n), jnp.float32),
        in_specs=[
            pl.BlockSpec(memory_space=pltpu.VMEM),
            pl.BlockSpec(memory_space=pltpu.VMEM),
        ],
        out_specs=pl.BlockSpec(memory_space=pltpu.VMEM),
        scratch_shapes=[
            pltpu.VMEM(B.shape, jnp.bfloat16),
            pltpu.VMEM((m, half), jnp.bfloat16),
            pltpu.VMEM((m, half), jnp.bfloat16),
            pltpu.VMEM((nhop, NP, piece, half), jnp.bfloat16),
            pltpu.VMEM((nhop, NP, piece, half), jnp.bfloat16),
            pltpu.VMEM((nhop, NP, piece, half), jnp.bfloat16),
            pltpu.VMEM((nhop, NP, piece, half), jnp.bfloat16),
            pltpu.VMEM((NP, piece, half), jnp.bfloat16),
            pltpu.VMEM((nhop, NP, piece, half), jnp.bfloat16),
            pltpu.VMEM((NP, piece, half), jnp.bfloat16),
            pltpu.VMEM((nhop, NP, piece, half), jnp.bfloat16),
            pltpu.SemaphoreType.DMA((nhop, NP)),
            pltpu.SemaphoreType.DMA((nhop, NP)),
            pltpu.SemaphoreType.DMA((nhop, NP)),
            pltpu.SemaphoreType.DMA((nhop, NP)),
            pltpu.SemaphoreType.DMA((nhop, NP)),
            pltpu.SemaphoreType.DMA((nhop, NP)),
            pltpu.SemaphoreType.DMA((nhop, NP)),
            pltpu.SemaphoreType.DMA((nhop, NP)),
        ],
        compiler_params=pltpu.CompilerParams(collective_id=0),
    )(A, B)


# device time: 50845 ns/iter; 1.0329x vs baseline; 1.0329x over previous
import jax
import jax.numpy as jnp
from jax import lax
from jax.experimental import pallas as pl
from jax.experimental.pallas import tpu as pltpu

N_DEV = 4
NP = 2


def kernel(A, B):
    m, _ = A.shape
    _, n = B.shape
    chunk = m // N_DEV
    half = n // 2
    piece = chunk // NP

    def body(
        a_ref,
        b_ref,
        out_ref,
        b_bf,
        pcL, pcR,
        rs_send_r, rs_recv_r, rs_send_l, rs_recv_l,
        rs_ssem_r, rs_rsem_r, rs_ssem_l, rs_rsem_l,
        ag_ssem_r, ag_rsem_r, ag_ssem_l, ag_rsem_l,
    ):
        my = lax.axis_index("i")
        left = (my - 1) % N_DEV
        right = (my + 1) % N_DEV

        barrier_sem = pltpu.get_barrier_semaphore()
        for nbr in (left, right):
            pl.semaphore_signal(
                barrier_sem,
                inc=1,
                device_id=(nbr,),
                device_id_type=pl.DeviceIdType.MESH,
            )
        pl.semaphore_wait(barrier_sem, 2)

        b_bf[...] = b_ref[...].astype(jnp.bfloat16)

        def rdma(src, dst, ssem, rsem, target):
            return pltpu.make_async_remote_copy(
                src_ref=src, dst_ref=dst, send_sem=ssem, recv_sem=rsem,
                device_id=(target,), device_id_type=pl.DeviceIdType.MESH,
            )

        rs_r = [[rdma(rs_send_r.at[s, p], rs_recv_r.at[s, p],
                      rs_ssem_r.at[s, p], rs_rsem_r.at[s, p], right)
                 for p in range(NP)] for s in range(3)]
        rs_l = [[rdma(rs_send_l.at[s, p], rs_recv_l.at[s, p],
                      rs_ssem_l.at[s, p], rs_rsem_l.at[s, p], left)
                 for p in range(NP)] for s in range(3)]

        def out_sl(c, p, col0):
            return out_ref.at[
                pl.ds(c * chunk + p * piece, piece), pl.ds(col0, half)
            ]

        ag_send_r = [[rdma(out_sl((my + 1 - s) % N_DEV, p, 0),
                           out_sl((my + 1 - s) % N_DEV, p, 0),
                           ag_ssem_r.at[s, p], ag_rsem_r.at[s, p], right)
                      for p in range(NP)] for s in range(3)]
        ag_recv_r = [[rdma(out_sl((my - s) % N_DEV, p, 0),
                           out_sl((my - s) % N_DEV, p, 0),
                           ag_ssem_r.at[s, p], ag_rsem_r.at[s, p], right)
                      for p in range(NP)] for s in range(3)]
        ag_send_l = [[rdma(out_sl((my - 1 + s) % N_DEV, p, half),
                           out_sl((my - 1 + s) % N_DEV, p, half),
                           ag_ssem_l.at[s, p], ag_rsem_l.at[s, p], left)
                      for p in range(NP)] for s in range(3)]
        ag_recv_l = [[rdma(out_sl((my + s) % N_DEV, p, half),
                           out_sl((my + s) % N_DEV, p, half),
                           ag_ssem_l.at[s, p], ag_rsem_l.at[s, p], left)
                      for p in range(NP)] for s in range(3)]

        for p in range(NP):
            ap = a_ref[
                pl.ds(my * chunk + p * piece, piece), :
            ].astype(jnp.bfloat16)
            rs_send_r[0, p, :, :] = jnp.dot(
                ap, b_bf[:, :half], preferred_element_type=jnp.float32
            ).astype(jnp.bfloat16)
            rs_r[0][p].start()
            rs_send_l[0, p, :, :] = jnp.dot(
                ap, b_bf[:, half:], preferred_element_type=jnp.float32
            ).astype(jnp.bfloat16)
            rs_l[0][p].start()

        for c in ((my - 1) % N_DEV, (my + 1) % N_DEV, (my + 2) % N_DEV):
            ac = a_ref[pl.ds(c * chunk, chunk), :].astype(jnp.bfloat16)
            pcL[pl.ds(c * chunk, chunk), :] = jnp.dot(
                ac, b_bf[:, :half], preferred_element_type=jnp.float32
            ).astype(jnp.bfloat16)
            pcR[pl.ds(c * chunk, chunk), :] = jnp.dot(
                ac, b_bf[:, half:], preferred_element_type=jnp.float32
            ).astype(jnp.bfloat16)

        for s in range(2):
            cr = (my - s - 1) % N_DEV
            cl = (my + s + 1) % N_DEV
            for p in range(NP):
                rs_r[s][p].wait_recv()
                rs_send_r[s + 1, p, :, :] = (
                    pcL[pl.ds(cr * chunk + p * piece, piece), :]
                    + rs_recv_r[s, p, :, :]
                )
                rs_r[s + 1][p].start()
                rs_l[s][p].wait_recv()
                rs_send_l[s + 1, p, :, :] = (
                    pcR[pl.ds(cl * chunk + p * piece, piece), :]
                    + rs_recv_l[s, p, :, :]
                )
                rs_l[s + 1][p].start()

        own_r = (my + 1) % N_DEV
        own_l = (my - 1) % N_DEV
        for p in range(NP):
            rs_r[2][p].wait_recv()
            zr = pcL[
                pl.ds(own_r * chunk + p * piece, piece), :
            ].astype(jnp.float32) + rs_recv_r[2, p, :, :].astype(jnp.float32)
            out_ref[
                pl.ds(own_r * chunk + p * piece, piece), pl.ds(0, half)
            ] = (zr / (1.0 + jnp.exp(-zr))).astype(jnp.bfloat16)
            ag_send_r[0][p].start()
            rs_l[2][p].wait_recv()
            zl = pcR[
                pl.ds(own_l * chunk + p * piece, piece), :
            ].astype(jnp.float32) + rs_recv_l[2, p, :, :].astype(jnp.float32)
            out_ref[
                pl.ds(own_l * chunk + p * piece, piece), pl.ds(half, half)
            ] = (zl / (1.0 + jnp.exp(-zl))).astype(jnp.bfloat16)
            ag_send_l[0][p].start()

        for s in range(3):
            for p in range(NP):
                ag_recv_r[s][p].wait_recv()
                if s < 2:
                    ag_send_r[s + 1][p].start()
                ag_recv_l[s][p].wait_recv()
                if s < 2:
                    ag_send_l[s + 1][p].start()

        for grid in (rs_r, rs_l, ag_send_r, ag_send_l):
            for ops in grid:
                for op in ops:
                    op.wait_send()

    nhop = N_DEV - 1
    return pl.pallas_call(
        body,
        out_shape=jax.ShapeDtypeStruct((m, n), jnp.bfloat16),
        in_specs=[
            pl.BlockSpec(memory_space=pltpu.VMEM),
            pl.BlockSpec(memory_space=pltpu.VMEM),
        ],
        out_specs=pl.BlockSpec(memory_space=pltpu.VMEM),
        scratch_shapes=[
            pltpu.VMEM(B.shape, jnp.bfloat16),
            pltpu.VMEM((m, half), jnp.bfloat16),
            pltpu.VMEM((m, half), jnp.bfloat16),
            pltpu.VMEM((nhop, NP, piece, half), jnp.bfloat16),
            pltpu.VMEM((nhop, NP, piece, half), jnp.bfloat16),
            pltpu.VMEM((nhop, NP, piece, half), jnp.bfloat16),
            pltpu.VMEM((nhop, NP, piece, half), jnp.bfloat16),
            pltpu.SemaphoreType.DMA((nhop, NP)),
            pltpu.SemaphoreType.DMA((nhop, NP)),
            pltpu.SemaphoreType.DMA((nhop, NP)),
            pltpu.SemaphoreType.DMA((nhop, NP)),
            pltpu.SemaphoreType.DMA((nhop, NP)),
            pltpu.SemaphoreType.DMA((nhop, NP)),
            pltpu.SemaphoreType.DMA((nhop, NP)),
            pltpu.SemaphoreType.DMA((nhop, NP)),
        ],
        compiler_params=pltpu.CompilerParams(collective_id=0),
    )(A, B)
